# baseline (speedup 1.0000x reference)
import functools

import jax
import jax.numpy as jnp
from jax import lax
from jax.experimental import pallas as pl
from jax.experimental.pallas import tpu as pltpu, tpu_sc as plsc

N_OUT = 100000
N_IN = 50000
D = 128
L = 16
NC = 2
NS = 16
NW = NC * NS
ROWS_PER_W = N_OUT // NW
WPAD = 3136
CHUNK = 128
NCHUNKS = WPAD // CHUNK + 1

_mesh = plsc.VectorSubcoreMesh(core_axis_name="c", subcore_axis_name="s")


@functools.partial(
    pl.kernel,
    out_type=jax.ShapeDtypeStruct((N_OUT, D), jnp.float32),
    mesh=_mesh,
    compiler_params=pltpu.CompilerParams(needs_layout_passes=False),
    scratch_types=[
        pltpu.VMEM((4000,), jnp.int32),
        pltpu.VMEM((WPAD,), jnp.int32),
        pltpu.VMEM((NCHUNKS, CHUNK), jnp.int32),
        pltpu.VMEM((NCHUNKS, CHUNK), jnp.int32),
        pltpu.VMEM((NCHUNKS, CHUNK), jnp.int32),
        pltpu.VMEM((5, CHUNK, D), jnp.float32),
        pltpu.VMEM((CHUNK, D), jnp.float32),
        pltpu.SemaphoreType.DMA((2,)),
        pltpu.SemaphoreType.DMA((5,)),
        pltpu.SemaphoreType.DMA((5,)),
        pltpu.SemaphoreType.DMA,
    ],
)
def _unpool_sc(h_hbm, idx_hbm, out_hbm, idx_v, winner, jw, gw, ju,
               rb, zerobuf, sem_i, sem_g, sem_s, sem_z):
    wid = lax.axis_index("s") * NC + lax.axis_index("c")
    lo = wid * ROWS_PER_W
    iota = lax.iota(jnp.int32, L)
    zeros_i = jnp.zeros((L,), jnp.int32)
    ones_i = jnp.ones((L,), jnp.int32)
    zeros_f = jnp.zeros((L,), jnp.float32)

    WIN = 2000
    NWIN = N_IN // WIN
    pltpu.async_copy(idx_hbm.at[pl.ds(0, WIN)], idx_v.at[pl.ds(0, WIN)], sem_i.at[0])
    pltpu.async_copy(idx_hbm.at[pl.ds(WIN, WIN)], idx_v.at[pl.ds(WIN, WIN)], sem_i.at[1])

    def _zw(k, c):
        winner[pl.ds(k * L, L)] = zeros_i
        return c
    lax.fori_loop(0, WPAD // L, _zw, 0)

    def _zb(t, c):
        zerobuf[t >> 3, pl.ds((t & 7) * L, L)] = zeros_f
        return c
    lax.fori_loop(0, CHUNK * D // L, _zb, 0)

    iota1 = iota + 1
    G = 5
    WVEC = WIN // L
    WGRP = WVEC // G

    def _compute(v, k):
        inr = (v >= lo) & (v < lo + ROWS_PER_W)
        return (v - lo, k * L + iota1, inr)

    def _scan(p, carry):
        w = p // WGRP
        pw = p - w * WGRP

        @pl.when(pw == 0)
        def _():
            @pl.when(w + 1 < NWIN)
            def _():
                pltpu.async_copy(idx_hbm.at[pl.ds((w + 1) * WIN, WIN)],
                                 idx_v.at[pl.ds(((w + 1) & 1) * WIN, WIN)],
                                 sem_i.at[(w + 1) & 1])
            pltpu.make_async_copy(idx_hbm.at[pl.ds(w * WIN, WIN)],
                                  idx_v.at[pl.ds((w & 1) * WIN, WIN)],
                                  sem_i.at[w & 1]).wait()
        base = (w & 1) * WIN
        vs = [idx_v[pl.ds(base + (G * pw + b) * L, L)] for b in range(G)]
        for b in range(G):
            lj, val, inr = carry[3 * b:3 * b + 3]
            plsc.store_scatter(winner, [lj], val, mask=inr)
        out = ()
        for b in range(G):
            out = out + _compute(vs[b], G * p + b)
        return out
    pltpu.make_async_copy(idx_hbm.at[pl.ds(0, WIN)],
                          idx_v.at[pl.ds(0, WIN)], sem_i.at[0]).wait()
    carry0 = ()
    for b in range(G):
        carry0 = carry0 + _compute(idx_v[pl.ds(b * L, L)], b)
    carry_z = lax.fori_loop(1, NWIN * WGRP, _scan, carry0)
    for b in range(G):
        lj, val, inr = carry_z[3 * b:3 * b + 3]
        plsc.store_scatter(winner, [lj], val, mask=inr)

    iotap = iota + 1

    def _compact(k, carry):
        nw, nu = carry
        w = winner[pl.ds(k * L, L)]
        pos = k * L + iota
        rem = ROWS_PER_W - k * L
        valid = pos < ROWS_PER_W
        jg = lo + pos
        mw = (w > 0) & valid
        mu = (w == 0) & valid
        cw = plsc.cumsum(jnp.where(mw, ones_i, zeros_i))
        pv = jnp.minimum(iotap, rem)
        cu = pv - cw
        offw = nw + cw - 1
        offu = nu + cu - 1
        plsc.store_scatter(jw, [offw >> 7, offw & 127], jg, mask=mw)
        plsc.store_scatter(gw, [offw >> 7, offw & 127], w - 1, mask=mw)
        plsc.store_scatter(ju, [offu >> 7, offu & 127], jg, mask=mu)
        tw = cw[15]
        vc = jnp.minimum(jnp.int32(L), rem)
        return nw + tw, nu + (vc - tw)
    nw, nu = lax.fori_loop(0, WPAD // L, _compact,
                           (jnp.int32(0), jnp.int32(0)))

    def _pad(buf, n):
        head = buf[0, pl.ds(0, L)]
        splat = head.at[zeros_i].get(mode="promise_in_bounds")
        last = ((n + CHUNK - 1) >> 7) - 1

        def _fix(t, c):
            base = last * CHUNK + t * L
            cur = buf[last, pl.ds(t * L, L)]
            buf[last, pl.ds(t * L, L)] = jnp.where(base + iota < n, cur, splat)
            return c

        @pl.when((n > 0) & ((n & 127) != 0))
        def _():
            lax.fori_loop(0, CHUNK // L, _fix, 0)

    _pad(ju, nu)
    nuc = (nu + CHUNK - 1) >> 7

    def _zfire(c, carry):
        pltpu.async_copy(zerobuf, out_hbm.at[ju.at[c]], sem_z)
        return carry
    lax.fori_loop(0, nuc, _zfire, 0)

    _pad(jw, nw)
    _pad(gw, nw)
    nwc = (nw + CHUNK - 1) >> 7

    for b in range(3):
        @pl.when(nwc > b)
        def _(b=b):
            pltpu.async_copy(h_hbm.at[gw.at[b]], rb.at[b], sem_g.at[b])

    def _move(c, carry):
        r = c % 5
        rn = (c + 3) % 5
        pltpu.make_async_copy(h_hbm.at[gw.at[c]], rb.at[r],
                              sem_g.at[r]).wait()

        @pl.when(c >= 2)
        def _():
            pltpu.make_async_copy(rb.at[rn], out_hbm.at[jw.at[c - 2]],
                                  sem_s.at[rn]).wait()

        @pl.when(c + 3 < nwc)
        def _():
            pltpu.async_copy(h_hbm.at[gw.at[c + 3]], rb.at[rn],
                             sem_g.at[rn])
        pltpu.async_copy(rb.at[r], out_hbm.at[jw.at[c]], sem_s.at[r])
        return carry
    lax.fori_loop(0, nwc, _move, 0)

    for b in range(2):
        cc = nwc - 2 + b

        @pl.when(cc >= 0)
        def _(cc=cc):
            pltpu.make_async_copy(rb.at[cc % 5], out_hbm.at[jw.at[cc]],
                                  sem_s.at[cc % 5]).wait()

    def _zdrain(c, carry):
        pltpu.make_async_copy(zerobuf, out_hbm.at[ju.at[c]], sem_z).wait()
        return carry
    lax.fori_loop(0, nuc, _zdrain, 0)


def kernel(node_nums, h, idx):
    return _unpool_sc(h, idx)


# speedup vs baseline: 1.1810x; 1.1810x over previous
import functools

import jax
import jax.numpy as jnp
from jax import lax
from jax.experimental import pallas as pl
from jax.experimental.pallas import tpu as pltpu, tpu_sc as plsc

N_OUT = 100000
N_IN = 50000
D = 128
L = 16
NC = 2
NS = 16
NW = NC * NS
ROWS_PER_W = N_OUT // NW
WPAD = 3136
CHUNK = 128
NCHUNKS = WPAD // CHUNK + 1

_mesh = plsc.VectorSubcoreMesh(core_axis_name="c", subcore_axis_name="s")


@functools.partial(
    pl.kernel,
    out_type=jax.ShapeDtypeStruct((N_OUT, D), jnp.float32),
    mesh=_mesh,
    compiler_params=pltpu.CompilerParams(needs_layout_passes=False),
    scratch_types=[
        pltpu.VMEM((N_IN,), jnp.int32),
        pltpu.VMEM((WPAD,), jnp.int32),
        pltpu.VMEM((NCHUNKS, CHUNK), jnp.int32),
        pltpu.VMEM((NCHUNKS, CHUNK), jnp.int32),
        pltpu.VMEM((NCHUNKS, CHUNK), jnp.int32),
        pltpu.VMEM((3, CHUNK, D), jnp.float32),
        pltpu.VMEM((CHUNK, D), jnp.float32),
        pltpu.SemaphoreType.DMA,
        pltpu.SemaphoreType.DMA((3,)),
        pltpu.SemaphoreType.DMA,
        pltpu.SemaphoreType.DMA,
    ],
)
def _unpool_sc(h_hbm, idx_hbm, out_hbm, idx_v, winner, jw, gw, ju,
               rb, zerobuf, sem_i, sem_g, sem_s, sem_z):
    wid = lax.axis_index("s") * NC + lax.axis_index("c")
    lo = wid * ROWS_PER_W
    iota = lax.iota(jnp.int32, L)
    zeros_i = jnp.zeros((L,), jnp.int32)
    ones_i = jnp.ones((L,), jnp.int32)
    zeros_f = jnp.zeros((L,), jnp.float32)

    idx_dma = pltpu.async_copy(idx_hbm, idx_v, sem_i)

    def _zw(k, c):
        winner[pl.ds(k * L, L)] = zeros_i
        return c
    lax.fori_loop(0, WPAD // L, _zw, 0)

    def _zb(t, c):
        zerobuf[t >> 3, pl.ds((t & 7) * L, L)] = zeros_f
        return c
    lax.fori_loop(0, CHUNK * D // L, _zb, 0)
    idx_dma.wait()

    iota1 = iota + 1
    NV = N_IN // L
    G = 5
    NP = NV // G

    def _compute(v, k):
        inr = (v >= lo) & (v < lo + ROWS_PER_W)
        return (v - lo, k * L + iota1, inr)

    def _scan(p, carry):
        vs = [idx_v[pl.ds((G * p + b) * L, L)] for b in range(G)]
        for b in range(G):
            lj, val, inr = carry[3 * b:3 * b + 3]
            plsc.store_scatter(winner, [lj], val, mask=inr)
        out = ()
        for b in range(G):
            out = out + _compute(vs[b], G * p + b)
        return out
    carry0 = ()
    for b in range(G):
        carry0 = carry0 + _compute(idx_v[pl.ds(b * L, L)], b)
    carry_z = lax.fori_loop(1, NP, _scan, carry0)
    for b in range(G):
        lj, val, inr = carry_z[3 * b:3 * b + 3]
        plsc.store_scatter(winner, [lj], val, mask=inr)

    iotap = iota + 1

    def _compact(k, carry):
        nw, nu = carry
        w = winner[pl.ds(k * L, L)]
        pos = k * L + iota
        rem = ROWS_PER_W - k * L
        valid = pos < ROWS_PER_W
        jg = lo + pos
        mw = (w > 0) & valid
        mu = (w == 0) & valid
        cw = plsc.cumsum(jnp.where(mw, ones_i, zeros_i))
        pv = jnp.minimum(iotap, rem)
        cu = pv - cw
        offw = nw + cw - 1
        offu = nu + cu - 1
        plsc.store_scatter(jw, [offw >> 7, offw & 127], jg, mask=mw)
        plsc.store_scatter(gw, [offw >> 7, offw & 127], w - 1, mask=mw)
        plsc.store_scatter(ju, [offu >> 7, offu & 127], jg, mask=mu)
        tw = cw[15]
        vc = jnp.minimum(jnp.int32(L), rem)
        return nw + tw, nu + (vc - tw)
    nw, nu = lax.fori_loop(0, WPAD // L, _compact,
                           (jnp.int32(0), jnp.int32(0)))

    def _pad(buf, n):
        head = buf[0, pl.ds(0, L)]
        splat = head.at[zeros_i].get(mode="promise_in_bounds")
        last = ((n + CHUNK - 1) >> 7) - 1

        def _fix(t, c):
            base = last * CHUNK + t * L
            cur = buf[last, pl.ds(t * L, L)]
            buf[last, pl.ds(t * L, L)] = jnp.where(base + iota < n, cur, splat)
            return c

        @pl.when((n > 0) & ((n & 127) != 0))
        def _():
            lax.fori_loop(0, CHUNK // L, _fix, 0)

    _pad(ju, nu)
    nuc = (nu + CHUNK - 1) >> 7

    def _zfire(c, carry):
        pltpu.async_copy(zerobuf, out_hbm.at[ju.at[c]], sem_z)
        return carry
    lax.fori_loop(0, nuc, _zfire, 0)

    _pad(jw, nw)
    _pad(gw, nw)
    nwc = (nw + CHUNK - 1) >> 7

    @pl.when(nwc > 0)
    def _():
        pltpu.async_copy(h_hbm.at[gw.at[0]], rb.at[0], sem_g.at[0])

    @pl.when(nwc > 1)
    def _():
        pltpu.async_copy(h_hbm.at[gw.at[1]], rb.at[1], sem_g.at[1])

    def _move(c, carry):
        r = c % 3
        rp = (c + 2) % 3
        pltpu.make_async_copy(h_hbm.at[gw.at[c]], rb.at[r],
                              sem_g.at[r]).wait()

        @pl.when(c > 0)
        def _():
            pltpu.make_async_copy(rb.at[rp], out_hbm.at[jw.at[c - 1]],
                                  sem_s).wait()

        @pl.when(c + 2 < nwc)
        def _():
            pltpu.async_copy(h_hbm.at[gw.at[c + 2]], rb.at[rp],
                             sem_g.at[rp])
        pltpu.async_copy(rb.at[r], out_hbm.at[jw.at[c]], sem_s)
        return carry
    lax.fori_loop(0, nwc, _move, 0)

    @pl.when(nwc > 0)
    def _():
        pltpu.make_async_copy(rb.at[(nwc - 1) % 3],
                              out_hbm.at[jw.at[nwc - 1]], sem_s).wait()

    def _zdrain(c, carry):
        pltpu.make_async_copy(zerobuf, out_hbm.at[ju.at[c]], sem_z).wait()
        return carry
    lax.fori_loop(0, nuc, _zdrain, 0)


def kernel(node_nums, h, idx):
    return _unpool_sc(h, idx)


# speedup vs baseline: 1.2062x; 1.0214x over previous
import functools

import jax
import jax.numpy as jnp
from jax import lax
from jax.experimental import pallas as pl
from jax.experimental.pallas import tpu as pltpu, tpu_sc as plsc

N_OUT = 100000
N_IN = 50000
D = 128
L = 16
NC = 2
NS = 16
NW = NC * NS
ROWS_PER_W = N_OUT // NW
WPAD = 3136
CHUNK = 128
NCHUNKS = WPAD // CHUNK + 1

_mesh = plsc.VectorSubcoreMesh(core_axis_name="c", subcore_axis_name="s")


@functools.partial(
    pl.kernel,
    out_type=jax.ShapeDtypeStruct((N_OUT, D), jnp.float32),
    mesh=_mesh,
    compiler_params=pltpu.CompilerParams(needs_layout_passes=False),
    scratch_types=[
        pltpu.VMEM((N_IN,), jnp.int32),
        pltpu.VMEM((WPAD,), jnp.int32),
        pltpu.VMEM((NCHUNKS, CHUNK), jnp.int32),
        pltpu.VMEM((NCHUNKS, CHUNK), jnp.int32),
        pltpu.VMEM((NCHUNKS, CHUNK), jnp.int32),
        pltpu.VMEM((3, CHUNK, D), jnp.float32),
        pltpu.VMEM((CHUNK, D), jnp.float32),
        pltpu.SemaphoreType.DMA,
        pltpu.SemaphoreType.DMA((3,)),
        pltpu.SemaphoreType.DMA,
        pltpu.SemaphoreType.DMA,
    ],
)
def _unpool_sc(h_hbm, idx_hbm, out_hbm, idx_v, winner, jw, gw, ju,
               rb, zerobuf, sem_i, sem_g, sem_s, sem_z):
    wid = lax.axis_index("s") * NC + lax.axis_index("c")
    lo = wid * ROWS_PER_W
    iota = lax.iota(jnp.int32, L)
    zeros_i = jnp.zeros((L,), jnp.int32)
    ones_i = jnp.ones((L,), jnp.int32)
    zeros_f = jnp.zeros((L,), jnp.float32)

    idx_dma = pltpu.async_copy(idx_hbm, idx_v, sem_i)

    def _zw(k, c):
        winner[pl.ds(k * L, L)] = zeros_i
        return c
    lax.fori_loop(0, WPAD // L, _zw, 0)

    def _zb(t, c):
        zerobuf[t >> 3, pl.ds((t & 7) * L, L)] = zeros_f
        return c
    lax.fori_loop(0, CHUNK * D // L, _zb, 0)
    idx_dma.wait()

    iota1 = iota + 1
    NV = N_IN // L
    G = 5
    NP = NV // G

    def _compute(v, k):
        inr = (v >= lo) & (v < lo + ROWS_PER_W)
        return (v - lo, k * L + iota1, inr)

    def _scan(p, carry):
        vs = [idx_v[pl.ds((G * p + b) * L, L)] for b in range(G)]
        for b in range(G):
            lj, val, inr = carry[3 * b:3 * b + 3]
            plsc.store_scatter(winner, [lj], val, mask=inr)
        out = ()
        for b in range(G):
            out = out + _compute(vs[b], G * p + b)
        return out
    carry0 = ()
    for b in range(G):
        carry0 = carry0 + _compute(idx_v[pl.ds(b * L, L)], b)
    carry_z = lax.fori_loop(1, NP, _scan, carry0)
    for b in range(G):
        lj, val, inr = carry_z[3 * b:3 * b + 3]
        plsc.store_scatter(winner, [lj], val, mask=inr)

    iotap = iota + 1

    def _compact(k, carry):
        nw, nu, zf, gf = carry
        w = winner[pl.ds(k * L, L)]
        pos = k * L + iota
        rem = ROWS_PER_W - k * L
        valid = pos < ROWS_PER_W
        jg = lo + pos
        mw = (w > 0) & valid
        mu = (w == 0) & valid
        cw = plsc.cumsum(jnp.where(mw, ones_i, zeros_i))
        pv = jnp.minimum(iotap, rem)
        cu = pv - cw
        offw = nw + cw - 1
        offu = nu + cu - 1
        plsc.store_scatter(jw, [offw >> 7, offw & 127], jg, mask=mw)
        plsc.store_scatter(gw, [offw >> 7, offw & 127], w - 1, mask=mw)
        plsc.store_scatter(ju, [offu >> 7, offu & 127], jg, mask=mu)
        tw = cw[15]
        vc = jnp.minimum(jnp.int32(L), rem)
        nw2, nu2 = nw + tw, nu + (vc - tw)

        @pl.when((nu2 >> 7) > zf)
        def _():
            pltpu.async_copy(zerobuf, out_hbm.at[ju.at[zf]], sem_z)

        @pl.when(((nw2 >> 7) > gf) & (gf < 2))
        def _():
            pltpu.async_copy(h_hbm.at[gw.at[gf]], rb.at[gf], sem_g.at[gf])
        zf2 = zf + jnp.where((nu2 >> 7) > zf, 1, 0)
        gf2 = gf + jnp.where(((nw2 >> 7) > gf) & (gf < 2), 1, 0)
        return nw2, nu2, zf2, gf2
    nw, nu, zf, gf = lax.fori_loop(
        0, WPAD // L, _compact,
        (jnp.int32(0), jnp.int32(0), jnp.int32(0), jnp.int32(0)))

    def _pad(buf, n):
        head = buf[0, pl.ds(0, L)]
        splat = head.at[zeros_i].get(mode="promise_in_bounds")
        last = ((n + CHUNK - 1) >> 7) - 1

        def _fix(t, c):
            base = last * CHUNK + t * L
            cur = buf[last, pl.ds(t * L, L)]
            buf[last, pl.ds(t * L, L)] = jnp.where(base + iota < n, cur, splat)
            return c

        @pl.when((n > 0) & ((n & 127) != 0))
        def _():
            lax.fori_loop(0, CHUNK // L, _fix, 0)

    _pad(ju, nu)
    nuc = (nu + CHUNK - 1) >> 7

    def _zfire(i, carry):
        pltpu.async_copy(zerobuf, out_hbm.at[ju.at[zf + i]], sem_z)
        return carry
    lax.fori_loop(0, nuc - zf, _zfire, 0)

    _pad(jw, nw)
    _pad(gw, nw)
    nwc = (nw + CHUNK - 1) >> 7

    @pl.when((nwc > 0) & (gf < 1))
    def _():
        pltpu.async_copy(h_hbm.at[gw.at[0]], rb.at[0], sem_g.at[0])

    @pl.when((nwc > 1) & (gf < 2))
    def _():
        pltpu.async_copy(h_hbm.at[gw.at[1]], rb.at[1], sem_g.at[1])

    def _move(c, carry):
        r = c % 3
        rp = (c + 2) % 3
        pltpu.make_async_copy(h_hbm.at[gw.at[c]], rb.at[r],
                              sem_g.at[r]).wait()

        @pl.when(c > 0)
        def _():
            pltpu.make_async_copy(rb.at[rp], out_hbm.at[jw.at[c - 1]],
                                  sem_s).wait()

        @pl.when(c + 2 < nwc)
        def _():
            pltpu.async_copy(h_hbm.at[gw.at[c + 2]], rb.at[rp],
                             sem_g.at[rp])
        pltpu.async_copy(rb.at[r], out_hbm.at[jw.at[c]], sem_s)
        return carry
    lax.fori_loop(0, nwc, _move, 0)

    @pl.when(nwc > 0)
    def _():
        pltpu.make_async_copy(rb.at[(nwc - 1) % 3],
                              out_hbm.at[jw.at[nwc - 1]], sem_s).wait()

    def _zdrain(c, carry):
        pltpu.make_async_copy(zerobuf, out_hbm.at[ju.at[c]], sem_z).wait()
        return carry
    lax.fori_loop(0, nuc, _zdrain, 0)


def kernel(node_nums, h, idx):
    return _unpool_sc(h, idx)
